# Initial kernel scaffold; baseline (speedup 1.0000x reference)
import jax
import jax.numpy as jnp
from jax.experimental import pallas as pl


def kernel(pred_scores, priors, pred_bboxes, gt_bboxes, gt_labels):
    raise NotImplementedError("write your pallas kernel here")


# speedup vs baseline: 1.6607x; 1.6607x over previous
import jax
import jax.numpy as jnp
from jax import lax
from jax.experimental import pallas as pl
from jax.experimental.pallas import tpu as pltpu

_EPS = 1e-07
_INF = 100000000.0
_RADIUS = 3.0
_IOU_W = 3.0
_K = 13
_N = 20000
_G = 128
_NC = 80
_R = 1024
_NP = 20480
_C = _NP // _R
_LN10 = 2.302585092994046
_IMAX = 2147483647
_MSK = -2147483648


def _body(ps_ref, pc_ref, st_ref, pb_ref, gb_ref, gl_ref, oh_ref,
          o_gt_ref, o_ov_ref, o_lb_ref,
          key_s, iou_s, cnt_s, gts_s, mpi_s, ra_s, ri_s):
    gt_i = lax.broadcasted_iota(jnp.int32, (_G, 1), 0)
    gt_f = gt_i.astype(jnp.float32)
    lane = lax.broadcasted_iota(jnp.int32, (1, _R), 1)

    gx1 = gb_ref[:, 0:1]
    gy1 = gb_ref[:, 1:2]
    gx2 = gb_ref[:, 2:3]
    gy2 = gb_ref[:, 3:4]
    pad = (gx1 + gy1 + gx2 + gy2) > 0.0
    gcx = (gx1 + gx2) * 0.5
    gcy = (gy1 + gy2) * 0.5
    garea = (gx2 - gx1) * (gy2 - gy1)
    gl_c = gl_ref[:, 0:1]
    oh = oh_ref[...]

    def p1(j, _):
        cx = pc_ref[j][0:1, :]
        cy = pc_ref[j][1:2, :]
        stride = st_ref[j][0:1, :]
        pb = pb_ref[j]
        px1 = pb[0:1, :]
        py1 = pb[1:2, :]
        px2 = pb[2:3, :]
        py2 = pb[3:4, :]

        mind = jnp.minimum(jnp.minimum(cx - gx1, cy - gy1),
                           jnp.minimum(gx2 - cx, gy2 - cy))
        inm = jnp.logical_and(mind > 0.0, pad)
        valid = jnp.sum(inm.astype(jnp.float32), axis=0, keepdims=True) > 0.0

        dx = cx - gcx
        dy = cy - gcy
        dist = jnp.sqrt(dx * dx + dy * dy) / stride
        dist = dist * valid.astype(jnp.float32)
        scp = jnp.exp(_LN10 * (dist - _RADIUS))

        parea = (px2 - px1) * (py2 - py1)
        iw = jnp.maximum(jnp.minimum(px2, gx2) - jnp.maximum(px1, gx1), 0.0)
        ih = jnp.maximum(jnp.minimum(py2, gy2) - jnp.maximum(py1, gy1), 0.0)
        ov = iw * ih
        iou = ov / jnp.maximum(parea + garea - ov, _EPS)

        pps = lax.dot_general(oh, ps_ref[j], (((1,), (0,)), ((), ())),
                              precision=lax.Precision.HIGHEST,
                              preferred_element_type=jnp.float32)
        sig = 1.0 / (1.0 + jnp.exp(-pps))
        sf = iou - sig
        bce = (jnp.maximum(pps, 0.0) - pps * iou
               + jnp.log1p(jnp.exp(-jnp.abs(pps))))
        cost = bce * (sf * sf) - jnp.log(iou + _EPS) * _IOU_W + scp
        cost = jnp.where(valid, cost, _INF)

        key = lax.bitcast_convert_type(cost, jnp.int32)
        key = jnp.where(lane + j * _R < _N, key, _IMAX)
        key_s[j] = key
        iou_s[j] = lax.bitcast_convert_type(iou, jnp.int32)

        rm = jnp.min(cost, axis=0, keepdims=True)
        ra = jnp.min(jnp.where(cost == rm, gt_i, _G), axis=0, keepdims=True)
        ra_s[j] = ra
        ri_s[j] = jnp.sum(jnp.where(gt_i == ra, iou, 0.0),
                          axis=0, keepdims=True)
        cnt_s[j] = jnp.zeros((1, _R), jnp.float32)
        gts_s[j] = jnp.zeros((1, _R), jnp.float32)
        mpi_s[j] = jnp.zeros((1, _R), jnp.float32)
        return 0

    lax.fori_loop(0, _C, p1, 0)

    def iou_t(t, acc):
        def cmax(j, m):
            return jnp.maximum(m, jnp.max(iou_s[j], axis=1, keepdims=True))
        colmax = lax.fori_loop(0, _C, cmax,
                               jnp.full((_G, 1), _MSK, jnp.int32))

        def rmin(j, r):
            cand = jnp.where(iou_s[j] == colmax, lane + j * _R, _NP)
            return jnp.minimum(r, jnp.min(cand, axis=1, keepdims=True))
        ridx = lax.fori_loop(0, _C, rmin, jnp.full((_G, 1), _NP, jnp.int32))

        def msk(j, _):
            ind = (lane + j * _R) == ridx
            iou_s[j] = jnp.where(ind, iou_s[j] | _MSK, iou_s[j])
            return 0
        lax.fori_loop(0, _C, msk, 0)
        return acc + lax.bitcast_convert_type(colmax, jnp.float32)

    acc = lax.fori_loop(0, _K, iou_t, jnp.zeros((_G, 1), jnp.float32))
    dk = jnp.maximum(acc.astype(jnp.int32), 1)

    def unmask(j, _):
        iou_s[j] = iou_s[j] & _IMAX
        return 0
    lax.fori_loop(0, _C, unmask, 0)

    def cost_t(t, _):
        def cmin(j, m):
            return jnp.minimum(m, jnp.min(key_s[j], axis=1, keepdims=True))
        colmin = lax.fori_loop(0, _C, cmin,
                               jnp.full((_G, 1), _IMAX, jnp.int32))

        def rmin(j, r):
            cand = jnp.where(key_s[j] == colmin, lane + j * _R, _NP)
            return jnp.minimum(r, jnp.min(cand, axis=1, keepdims=True))
        ridx = lax.fori_loop(0, _C, rmin, jnp.full((_G, 1), _NP, jnp.int32))

        sel = t < dk

        def upd(j, _):
            ind = (lane + j * _R) == ridx
            key_s[j] = jnp.where(ind, _IMAX, key_s[j])
            msel = jnp.logical_and(ind, sel)
            mf = msel.astype(jnp.float32)
            cnt_s[j] = cnt_s[j] + jnp.sum(mf, axis=0, keepdims=True)
            gts_s[j] = gts_s[j] + jnp.sum(mf * gt_f, axis=0, keepdims=True)
            iouf = lax.bitcast_convert_type(iou_s[j], jnp.float32)
            mpi_s[j] = mpi_s[j] + jnp.sum(jnp.where(msel, iouf, 0.0),
                                          axis=0, keepdims=True)
            return 0
        lax.fori_loop(0, _C, upd, 0)
        return 0

    lax.fori_loop(0, _K, cost_t, 0)

    def fin(j, _):
        c = cnt_s[j]
        fg = c > 0.5
        multi = c > 1.5
        mg = jnp.where(multi, ra_s[j], gts_s[j].astype(jnp.int32))
        ovl = jnp.where(multi, ri_s[j], mpi_s[j])
        lab = jnp.sum(jnp.where(gt_i == mg, gl_c, 0), axis=0, keepdims=True)
        o_gt_ref[j] = jnp.where(fg, mg + 1, 0)
        o_ov_ref[j] = jnp.where(fg, ovl, -_INF)
        o_lb_ref[j] = jnp.where(fg, lab, -1)
        return 0

    lax.fori_loop(0, _C, fin, 0)


def kernel(pred_scores, priors, pred_bboxes, gt_bboxes, gt_labels):
    padn = _NP - _N

    def chunked(xt):
        xt = jnp.pad(xt, ((0, 0), (0, padn)))
        k = xt.shape[0]
        return xt.reshape(k, _C, _R).transpose(1, 0, 2)

    ps = chunked(pred_scores.T)
    pc = chunked(priors[:, :2].T)
    st = chunked(priors[:, 2:3].T)
    pb = chunked(pred_bboxes.T)
    gl_c = gt_labels.reshape(_G, 1)
    oh = (gl_c == jnp.arange(_NC, dtype=jnp.int32)[None, :]).astype(jnp.float32)

    out_shape = [
        jax.ShapeDtypeStruct((_C, 1, _R), jnp.int32),
        jax.ShapeDtypeStruct((_C, 1, _R), jnp.float32),
        jax.ShapeDtypeStruct((_C, 1, _R), jnp.int32),
    ]
    scratch = [
        pltpu.VMEM((_C, _G, _R), jnp.int32),
        pltpu.VMEM((_C, _G, _R), jnp.int32),
        pltpu.VMEM((_C, 1, _R), jnp.float32),
        pltpu.VMEM((_C, 1, _R), jnp.float32),
        pltpu.VMEM((_C, 1, _R), jnp.float32),
        pltpu.VMEM((_C, 1, _R), jnp.int32),
        pltpu.VMEM((_C, 1, _R), jnp.float32),
    ]
    a, o, l = pl.pallas_call(
        _body,
        out_shape=out_shape,
        scratch_shapes=scratch,
    )(ps, pc, st, pb, gt_bboxes, gl_c, oh)
    return (a.reshape(_NP)[:_N], o.reshape(_NP)[:_N], l.reshape(_NP)[:_N])


# speedup vs baseline: 1.8363x; 1.1057x over previous
import jax
import jax.numpy as jnp
from jax import lax
from jax.experimental import pallas as pl
from jax.experimental.pallas import tpu as pltpu

_EPS = 1e-07
_INF = 100000000.0
_RADIUS = 3.0
_IOU_W = 3.0
_K = 13
_N = 20000
_G = 128
_NC = 80
_R = 1024
_NP = 20480
_C = _NP // _R
_LN10 = 2.302585092994046
_IMAX = 2147483647
_MSK = -2147483648


def _body(ps_ref, pc_ref, st_ref, pb_ref, gb_ref, gl_ref, oh_ref,
          o_gt_ref, o_ov_ref, o_lb_ref,
          key_s, iou_s, cnt_s, mpi_s, ra_s, ri_s):
    gt_i = lax.broadcasted_iota(jnp.int32, (_G, 1), 0)
    lane = lax.broadcasted_iota(jnp.int32, (1, _R), 1)

    gx1 = gb_ref[:, 0:1]
    gy1 = gb_ref[:, 1:2]
    gx2 = gb_ref[:, 2:3]
    gy2 = gb_ref[:, 3:4]
    pad = (gx1 + gy1 + gx2 + gy2) > 0.0
    gcx = (gx1 + gx2) * 0.5
    gcy = (gy1 + gy2) * 0.5
    garea = (gx2 - gx1) * (gy2 - gy1)
    gl_c = gl_ref[:, 0:1]
    oh = oh_ref[...]

    def p1(j, _):
        cx = pc_ref[j][0:1, :]
        cy = pc_ref[j][1:2, :]
        stride = st_ref[j][0:1, :]
        pb = pb_ref[j]
        px1 = pb[0:1, :]
        py1 = pb[1:2, :]
        px2 = pb[2:3, :]
        py2 = pb[3:4, :]

        mind = jnp.minimum(jnp.minimum(cx - gx1, cy - gy1),
                           jnp.minimum(gx2 - cx, gy2 - cy))
        inm = jnp.logical_and(mind > 0.0, pad)
        valid = jnp.sum(inm.astype(jnp.float32), axis=0, keepdims=True) > 0.0

        dx = cx - gcx
        dy = cy - gcy
        dist = jnp.sqrt(dx * dx + dy * dy) / stride
        dist = dist * valid.astype(jnp.float32)
        scp = jnp.exp(_LN10 * (dist - _RADIUS))

        parea = (px2 - px1) * (py2 - py1)
        iw = jnp.maximum(jnp.minimum(px2, gx2) - jnp.maximum(px1, gx1), 0.0)
        ih = jnp.maximum(jnp.minimum(py2, gy2) - jnp.maximum(py1, gy1), 0.0)
        ov = iw * ih
        iou = ov / jnp.maximum(parea + garea - ov, _EPS)

        pps = lax.dot_general(oh, ps_ref[j], (((1,), (0,)), ((), ())),
                              precision=lax.Precision.HIGHEST,
                              preferred_element_type=jnp.float32)
        sig = 1.0 / (1.0 + jnp.exp(-pps))
        sf = iou - sig
        bce = (jnp.maximum(pps, 0.0) - pps * iou
               + jnp.log1p(jnp.exp(-jnp.abs(pps))))
        cost = bce * (sf * sf) - jnp.log(iou + _EPS) * _IOU_W + scp
        cost = jnp.where(valid, cost, _INF)

        key = lax.bitcast_convert_type(cost, jnp.int32)
        key = jnp.where(lane + j * _R < _N, key, _IMAX)
        key_s[j] = key
        iou_s[j] = lax.bitcast_convert_type(iou, jnp.int32)

        rm = jnp.min(cost, axis=0, keepdims=True)
        ra = jnp.min(jnp.where(cost == rm, gt_i, _G), axis=0, keepdims=True)
        ra_s[j] = ra
        ri_s[j] = jnp.sum(jnp.where(gt_i == ra, iou, 0.0),
                          axis=0, keepdims=True)
        cnt_s[j] = jnp.zeros((1, _R), jnp.int32)
        mpi_s[j] = jnp.zeros((1, _R), jnp.float32)
        return 0

    lax.fori_loop(0, _C, p1, 0)

    def iou_t(t, carry):
        acc, taken = carry

        def cmax(j, m):
            return jnp.maximum(m, jnp.max(iou_s[j], axis=1, keepdims=True))
        colmax = lax.fori_loop(0, _C, cmax,
                               jnp.full((_G, 1), _MSK, jnp.int32))

        def msk(j, c):
            eqm = iou_s[j] == colmax
            iou_s[j] = jnp.where(eqm, iou_s[j] | _MSK, iou_s[j])
            return c + jnp.sum(eqm.astype(jnp.int32), axis=1, keepdims=True)
        count = lax.fori_loop(0, _C, msk, jnp.zeros((_G, 1), jnp.int32))

        m = jnp.minimum(count, jnp.maximum(_K - taken, 0))
        v = jnp.maximum(lax.bitcast_convert_type(colmax, jnp.float32), 0.0)
        return (acc + v * m.astype(jnp.float32), taken + m)

    acc, _ = lax.fori_loop(0, _K, iou_t,
                           (jnp.zeros((_G, 1), jnp.float32),
                            jnp.zeros((_G, 1), jnp.int32)))
    dk = jnp.maximum(acc.astype(jnp.int32), 1)

    def unmask(j, _):
        iou_s[j] = iou_s[j] & _IMAX
        return 0
    lax.fori_loop(0, _C, unmask, 0)

    def cost_t(t, _):
        def cmin(j, m):
            return jnp.minimum(m, jnp.min(key_s[j], axis=1, keepdims=True))
        colmin = lax.fori_loop(0, _C, cmin,
                               jnp.full((_G, 1), _IMAX, jnp.int32))

        def rmin(j, r):
            cand = jnp.where(key_s[j] == colmin, lane + j * _R, _NP)
            return jnp.minimum(r, jnp.min(cand, axis=1, keepdims=True))
        ridx = lax.fori_loop(0, _C, rmin, jnp.full((_G, 1), _NP, jnp.int32))

        sel = t < dk
        gpack = jnp.where(sel, 2048 + gt_i, 0)

        def upd(j, _):
            ind = (lane + j * _R) == ridx
            key_s[j] = jnp.where(ind, _IMAX, key_s[j])
            cnt_s[j] = cnt_s[j] + jnp.sum(jnp.where(ind, gpack, 0),
                                          axis=0, keepdims=True)
            iouf = lax.bitcast_convert_type(iou_s[j], jnp.float32)
            msel = jnp.logical_and(ind, sel)
            mpi_s[j] = mpi_s[j] + jnp.sum(jnp.where(msel, iouf, 0.0),
                                          axis=0, keepdims=True)
            return 0
        lax.fori_loop(0, _C, upd, 0)
        return 0

    lax.fori_loop(0, _K, cost_t, 0)

    def fin(j, _):
        a = cnt_s[j]
        c = a >> 11
        fg = c > 0
        multi = c > 1
        mg = jnp.where(multi, ra_s[j], a & 2047)
        ovl = jnp.where(multi, ri_s[j], mpi_s[j])
        lab = jnp.sum(jnp.where(gt_i == mg, gl_c, 0), axis=0, keepdims=True)
        o_gt_ref[j] = jnp.where(fg, mg + 1, 0)
        o_ov_ref[j] = jnp.where(fg, ovl, -_INF)
        o_lb_ref[j] = jnp.where(fg, lab, -1)
        return 0

    lax.fori_loop(0, _C, fin, 0)


def kernel(pred_scores, priors, pred_bboxes, gt_bboxes, gt_labels):
    padn = _NP - _N

    def chunked(xt):
        xt = jnp.pad(xt, ((0, 0), (0, padn)))
        k = xt.shape[0]
        return xt.reshape(k, _C, _R).transpose(1, 0, 2)

    ps = chunked(pred_scores.T)
    pc = chunked(priors[:, :2].T)
    st = chunked(priors[:, 2:3].T)
    pb = chunked(pred_bboxes.T)
    gl_c = gt_labels.reshape(_G, 1)
    oh = (gl_c == jnp.arange(_NC, dtype=jnp.int32)[None, :]).astype(jnp.float32)

    out_shape = [
        jax.ShapeDtypeStruct((_C, 1, _R), jnp.int32),
        jax.ShapeDtypeStruct((_C, 1, _R), jnp.float32),
        jax.ShapeDtypeStruct((_C, 1, _R), jnp.int32),
    ]
    scratch = [
        pltpu.VMEM((_C, _G, _R), jnp.int32),
        pltpu.VMEM((_C, _G, _R), jnp.int32),
        pltpu.VMEM((_C, 1, _R), jnp.int32),
        pltpu.VMEM((_C, 1, _R), jnp.float32),
        pltpu.VMEM((_C, 1, _R), jnp.int32),
        pltpu.VMEM((_C, 1, _R), jnp.float32),
    ]
    a, o, l = pl.pallas_call(
        _body,
        out_shape=out_shape,
        scratch_shapes=scratch,
    )(ps, pc, st, pb, gt_bboxes, gl_c, oh)
    return (a.reshape(_NP)[:_N], o.reshape(_NP)[:_N], l.reshape(_NP)[:_N])


# speedup vs baseline: 2.2159x; 1.2067x over previous
import jax
import jax.numpy as jnp
from jax import lax
from jax.experimental import pallas as pl
from jax.experimental.pallas import tpu as pltpu

_EPS = 1e-07
_INF = 100000000.0
_RADIUS = 3.0
_IOU_W = 3.0
_K = 13
_N = 20000
_G = 128
_NC = 80
_R = 1024
_NP = 20480
_C = _NP // _R
_LN10 = 2.302585092994046
_IMAX = 2147483647
_MSK = -2147483648


def _body(ps_ref, pc_ref, st_ref, pb_ref, gb_ref, gl_ref, oh_ref,
          o_gt_ref, o_ov_ref, o_lb_ref,
          key_s, iou_s, cnt_s, mpi_s, ra_s, ri_s):
    gt_i = lax.broadcasted_iota(jnp.int32, (_G, 1), 0)
    lane = lax.broadcasted_iota(jnp.int32, (1, _R), 1)

    gx1 = gb_ref[:, 0:1]
    gy1 = gb_ref[:, 1:2]
    gx2 = gb_ref[:, 2:3]
    gy2 = gb_ref[:, 3:4]
    pad = (gx1 + gy1 + gx2 + gy2) > 0.0
    gcx = (gx1 + gx2) * 0.5
    gcy = (gy1 + gy2) * 0.5
    garea = (gx2 - gx1) * (gy2 - gy1)
    gl_c = gl_ref[:, 0:1]
    oh = oh_ref[...]

    def p1(j, _):
        cx = pc_ref[j][0:1, :]
        cy = pc_ref[j][1:2, :]
        stride = st_ref[j][0:1, :]
        pb = pb_ref[j]
        px1 = pb[0:1, :]
        py1 = pb[1:2, :]
        px2 = pb[2:3, :]
        py2 = pb[3:4, :]

        mind = jnp.minimum(jnp.minimum(cx - gx1, cy - gy1),
                           jnp.minimum(gx2 - cx, gy2 - cy))
        inm = jnp.logical_and(mind > 0.0, pad)
        valid = jnp.sum(inm.astype(jnp.float32), axis=0, keepdims=True) > 0.0

        dx = cx - gcx
        dy = cy - gcy
        dist = jnp.sqrt(dx * dx + dy * dy) / stride
        dist = dist * valid.astype(jnp.float32)
        scp = jnp.exp(_LN10 * (dist - _RADIUS))

        parea = (px2 - px1) * (py2 - py1)
        iw = jnp.maximum(jnp.minimum(px2, gx2) - jnp.maximum(px1, gx1), 0.0)
        ih = jnp.maximum(jnp.minimum(py2, gy2) - jnp.maximum(py1, gy1), 0.0)
        ov = iw * ih
        iou = ov / jnp.maximum(parea + garea - ov, _EPS)

        pps = lax.dot_general(oh, ps_ref[j], (((1,), (0,)), ((), ())),
                              precision=lax.Precision.HIGHEST,
                              preferred_element_type=jnp.float32)
        sig = 1.0 / (1.0 + jnp.exp(-pps))
        sf = iou - sig
        bce = (jnp.maximum(pps, 0.0) - pps * iou
               + jnp.log1p(jnp.exp(-jnp.abs(pps))))
        cost = bce * (sf * sf) - jnp.log(iou + _EPS) * _IOU_W + scp
        cost = jnp.where(valid, cost, _INF)

        key = lax.bitcast_convert_type(cost, jnp.int32)
        key = jnp.where(lane + j * _R < _N, key, _IMAX)
        key_s[j] = key
        iou_s[j] = lax.bitcast_convert_type(iou, jnp.int32)

        rm = jnp.min(cost, axis=0, keepdims=True)
        ra = jnp.min(jnp.where(cost == rm, gt_i, _G), axis=0, keepdims=True)
        ra_s[j] = ra
        ri_s[j] = jnp.sum(jnp.where(gt_i == ra, iou, 0.0),
                          axis=0, keepdims=True)
        cnt_s[j] = jnp.zeros((1, _R), jnp.int32)
        mpi_s[j] = jnp.zeros((1, _R), jnp.float32)
        return 0

    lax.fori_loop(0, _C, p1, 0)

    def iou_t(t, carry):
        acc, taken = carry

        def cmax(j, m):
            return jnp.maximum(m, jnp.max(iou_s[j], axis=1, keepdims=True))
        colmax = lax.fori_loop(0, _C, cmax,
                               jnp.full((_G, 1), _MSK, jnp.int32))

        def msk(j, c):
            eqm = iou_s[j] == colmax
            iou_s[j] = jnp.where(eqm, iou_s[j] | _MSK, iou_s[j])
            return c + jnp.sum(eqm.astype(jnp.int32), axis=1, keepdims=True)
        count = lax.fori_loop(0, _C, msk, jnp.zeros((_G, 1), jnp.int32))

        m = jnp.minimum(count, jnp.maximum(_K - taken, 0))
        v = jnp.maximum(lax.bitcast_convert_type(colmax, jnp.float32), 0.0)
        return (acc + v * m.astype(jnp.float32), taken + m)

    acc, _ = lax.fori_loop(0, _K, iou_t,
                           (jnp.zeros((_G, 1), jnp.float32),
                            jnp.zeros((_G, 1), jnp.int32)))
    dk = jnp.maximum(acc.astype(jnp.int32), 1)

    def unmask(j, _):
        iou_s[j] = iou_s[j] & _IMAX
        return 0
    lax.fori_loop(0, _C, unmask, 0)

    def cost_t(t, _):
        def cmin(j, m):
            return jnp.minimum(m, jnp.min(key_s[j], axis=1, keepdims=True))
        colmin = lax.fori_loop(0, _C, cmin,
                               jnp.full((_G, 1), _IMAX, jnp.int32))

        def rmin(j, r):
            cand = jnp.where(key_s[j] == colmin, lane + j * _R, _NP)
            return jnp.minimum(r, jnp.min(cand, axis=1, keepdims=True))
        ridx = lax.fori_loop(0, _C, rmin, jnp.full((_G, 1), _NP, jnp.int32))

        sel = t < dk
        gpack = jnp.where(sel, 2048 + gt_i, 0)

        def upd(j, _):
            ind = (lane + j * _R) == ridx
            key_s[j] = jnp.where(ind, _IMAX, key_s[j])
            cnt_s[j] = cnt_s[j] + jnp.sum(jnp.where(ind, gpack, 0),
                                          axis=0, keepdims=True)
            iouf = lax.bitcast_convert_type(iou_s[j], jnp.float32)
            msel = jnp.logical_and(ind, sel)
            mpi_s[j] = mpi_s[j] + jnp.sum(jnp.where(msel, iouf, 0.0),
                                          axis=0, keepdims=True)
            return 0
        lax.fori_loop(0, _C, upd, 0)
        return 0

    lax.fori_loop(0, jnp.max(dk), cost_t, 0)

    def fin(j, _):
        a = cnt_s[j]
        c = a >> 11
        fg = c > 0
        multi = c > 1
        mg = jnp.where(multi, ra_s[j], a & 2047)
        ovl = jnp.where(multi, ri_s[j], mpi_s[j])
        lab = jnp.sum(jnp.where(gt_i == mg, gl_c, 0), axis=0, keepdims=True)
        o_gt_ref[j] = jnp.where(fg, mg + 1, 0)
        o_ov_ref[j] = jnp.where(fg, ovl, -_INF)
        o_lb_ref[j] = jnp.where(fg, lab, -1)
        return 0

    lax.fori_loop(0, _C, fin, 0)


def kernel(pred_scores, priors, pred_bboxes, gt_bboxes, gt_labels):
    padn = _NP - _N

    def chunked(xt):
        xt = jnp.pad(xt, ((0, 0), (0, padn)))
        k = xt.shape[0]
        return xt.reshape(k, _C, _R).transpose(1, 0, 2)

    ps = chunked(pred_scores.T)
    pc = chunked(priors[:, :2].T)
    st = chunked(priors[:, 2:3].T)
    pb = chunked(pred_bboxes.T)
    gl_c = gt_labels.reshape(_G, 1)
    oh = (gl_c == jnp.arange(_NC, dtype=jnp.int32)[None, :]).astype(jnp.float32)

    out_shape = [
        jax.ShapeDtypeStruct((_C, 1, _R), jnp.int32),
        jax.ShapeDtypeStruct((_C, 1, _R), jnp.float32),
        jax.ShapeDtypeStruct((_C, 1, _R), jnp.int32),
    ]
    scratch = [
        pltpu.VMEM((_C, _G, _R), jnp.int32),
        pltpu.VMEM((_C, _G, _R), jnp.int32),
        pltpu.VMEM((_C, 1, _R), jnp.int32),
        pltpu.VMEM((_C, 1, _R), jnp.float32),
        pltpu.VMEM((_C, 1, _R), jnp.int32),
        pltpu.VMEM((_C, 1, _R), jnp.float32),
    ]
    a, o, l = pl.pallas_call(
        _body,
        out_shape=out_shape,
        scratch_shapes=scratch,
    )(ps, pc, st, pb, gt_bboxes, gl_c, oh)
    return (a.reshape(_NP)[:_N], o.reshape(_NP)[:_N], l.reshape(_NP)[:_N])


# speedup vs baseline: 2.4361x; 1.0994x over previous
import jax
import jax.numpy as jnp
from jax import lax
from jax.experimental import pallas as pl
from jax.experimental.pallas import tpu as pltpu

_EPS = 1e-07
_INF = 100000000.0
_RADIUS = 3.0
_IOU_W = 3.0
_K = 13
_N = 20000
_G = 128
_NC = 80
_R = 1024
_NP = 20480
_C = _NP // _R
_LN10 = 2.302585092994046
_IMAX = 2147483647
_MSK = -2147483648


def _body(ps_ref, pc_ref, st_ref, pb_ref, gb_ref, gl_ref, oh_ref,
          o_gt_ref, o_ov_ref, o_lb_ref,
          key_s, iou_s, cnt_s, mpi_s, ra_s, ri_s):
    gt_i = lax.broadcasted_iota(jnp.int32, (_G, 1), 0)
    lane = lax.broadcasted_iota(jnp.int32, (1, _R), 1)

    gx1 = gb_ref[:, 0:1]
    gy1 = gb_ref[:, 1:2]
    gx2 = gb_ref[:, 2:3]
    gy2 = gb_ref[:, 3:4]
    pad = (gx1 + gy1 + gx2 + gy2) > 0.0
    gcx = (gx1 + gx2) * 0.5
    gcy = (gy1 + gy2) * 0.5
    garea = (gx2 - gx1) * (gy2 - gy1)
    gl_c = gl_ref[:, 0:1]
    oh = oh_ref[...]

    def p1(j, _):
        cx = pc_ref[j][0:1, :]
        cy = pc_ref[j][1:2, :]
        stride = st_ref[j][0:1, :]
        pb = pb_ref[j]
        px1 = pb[0:1, :]
        py1 = pb[1:2, :]
        px2 = pb[2:3, :]
        py2 = pb[3:4, :]

        mind = jnp.minimum(jnp.minimum(cx - gx1, cy - gy1),
                           jnp.minimum(gx2 - cx, gy2 - cy))
        inm = jnp.logical_and(mind > 0.0, pad)
        valid = jnp.sum(inm.astype(jnp.float32), axis=0, keepdims=True) > 0.0

        dx = cx - gcx
        dy = cy - gcy
        dist = jnp.sqrt(dx * dx + dy * dy) / stride
        dist = dist * valid.astype(jnp.float32)
        scp = jnp.exp(_LN10 * (dist - _RADIUS))

        parea = (px2 - px1) * (py2 - py1)
        iw = jnp.maximum(jnp.minimum(px2, gx2) - jnp.maximum(px1, gx1), 0.0)
        ih = jnp.maximum(jnp.minimum(py2, gy2) - jnp.maximum(py1, gy1), 0.0)
        ov = iw * ih
        iou = ov / jnp.maximum(parea + garea - ov, _EPS)

        pps = lax.dot_general(oh, ps_ref[j], (((1,), (0,)), ((), ())),
                              precision=lax.Precision.HIGHEST,
                              preferred_element_type=jnp.float32)
        sig = 1.0 / (1.0 + jnp.exp(-pps))
        sf = iou - sig
        bce = (jnp.maximum(pps, 0.0) - pps * iou
               + jnp.log1p(jnp.exp(-jnp.abs(pps))))
        cost = bce * (sf * sf) - jnp.log(iou + _EPS) * _IOU_W + scp
        cost = jnp.where(valid, cost, _INF)

        key = lax.bitcast_convert_type(cost, jnp.int32)
        key = jnp.where(lane + j * _R < _N, key, _IMAX)
        key_s[j] = key
        iou_s[j] = lax.bitcast_convert_type(iou, jnp.int32)

        rm = jnp.min(cost, axis=0, keepdims=True)
        ra = jnp.min(jnp.where(cost == rm, gt_i, _G), axis=0, keepdims=True)
        ra_s[j] = ra
        ri_s[j] = jnp.sum(jnp.where(gt_i == ra, iou, 0.0),
                          axis=0, keepdims=True)
        cnt_s[j] = jnp.zeros((1, _R), jnp.int32)
        mpi_s[j] = jnp.zeros((1, _R), jnp.float32)
        return 0

    lax.fori_loop(0, _C, p1, 0)

    def cmax0(j, m):
        return jnp.maximum(m, jnp.max(iou_s[j], axis=1, keepdims=True))
    colmax0 = lax.fori_loop(0, _C, cmax0, jnp.full((_G, 1), _MSK, jnp.int32))

    def iou_t(t, carry):
        acc, taken, colmax = carry

        def msk(j, cm):
            c, m = cm
            x = iou_s[j]
            eqm = x == colmax
            x2 = jnp.where(eqm, x | _MSK, x)
            iou_s[j] = x2
            c = c + jnp.sum(eqm.astype(jnp.int32), axis=1, keepdims=True)
            m = jnp.maximum(m, jnp.max(x2, axis=1, keepdims=True))
            return (c, m)
        count, nxt = lax.fori_loop(0, _C, msk,
                                   (jnp.zeros((_G, 1), jnp.int32),
                                    jnp.full((_G, 1), _MSK, jnp.int32)))

        m = jnp.minimum(count, jnp.maximum(_K - taken, 0))
        v = jnp.maximum(lax.bitcast_convert_type(colmax, jnp.float32), 0.0)
        return (acc + v * m.astype(jnp.float32), taken + m, nxt)

    acc, _, _ = lax.fori_loop(0, _K, iou_t,
                              (jnp.zeros((_G, 1), jnp.float32),
                               jnp.zeros((_G, 1), jnp.int32), colmax0))
    dk = jnp.maximum(acc.astype(jnp.int32), 1)

    def unmask(j, _):
        iou_s[j] = iou_s[j] & _IMAX
        return 0
    lax.fori_loop(0, _C, unmask, 0)

    def cmin0(j, m):
        return jnp.minimum(m, jnp.min(key_s[j], axis=1, keepdims=True))
    colmin0 = lax.fori_loop(0, _C, cmin0, jnp.full((_G, 1), _IMAX, jnp.int32))

    def cost_t(t, colmin):
        def rmin(j, r):
            cand = jnp.where(key_s[j] == colmin, lane + j * _R, _NP)
            return jnp.minimum(r, jnp.min(cand, axis=1, keepdims=True))
        ridx = lax.fori_loop(0, _C, rmin, jnp.full((_G, 1), _NP, jnp.int32))

        sel = t < dk
        gpack = jnp.where(sel, 2048 + gt_i, 0)

        def upd(j, m):
            ind = (lane + j * _R) == ridx
            k2 = jnp.where(ind, _IMAX, key_s[j])
            key_s[j] = k2
            cnt_s[j] = cnt_s[j] + jnp.sum(jnp.where(ind, gpack, 0),
                                          axis=0, keepdims=True)
            iouf = lax.bitcast_convert_type(iou_s[j], jnp.float32)
            msel = jnp.logical_and(ind, sel)
            mpi_s[j] = mpi_s[j] + jnp.sum(jnp.where(msel, iouf, 0.0),
                                          axis=0, keepdims=True)
            return jnp.minimum(m, jnp.min(k2, axis=1, keepdims=True))
        return lax.fori_loop(0, _C, upd, jnp.full((_G, 1), _IMAX, jnp.int32))

    lax.fori_loop(0, jnp.max(dk), cost_t, colmin0)

    def fin(j, _):
        a = cnt_s[j]
        c = a >> 11
        fg = c > 0
        multi = c > 1
        mg = jnp.where(multi, ra_s[j], a & 2047)
        ovl = jnp.where(multi, ri_s[j], mpi_s[j])
        lab = jnp.sum(jnp.where(gt_i == mg, gl_c, 0), axis=0, keepdims=True)
        o_gt_ref[j] = jnp.where(fg, mg + 1, 0)
        o_ov_ref[j] = jnp.where(fg, ovl, -_INF)
        o_lb_ref[j] = jnp.where(fg, lab, -1)
        return 0

    lax.fori_loop(0, _C, fin, 0)


def kernel(pred_scores, priors, pred_bboxes, gt_bboxes, gt_labels):
    padn = _NP - _N

    def chunked(xt):
        xt = jnp.pad(xt, ((0, 0), (0, padn)))
        k = xt.shape[0]
        return xt.reshape(k, _C, _R).transpose(1, 0, 2)

    ps = chunked(pred_scores.T)
    pc = chunked(priors[:, :2].T)
    st = chunked(priors[:, 2:3].T)
    pb = chunked(pred_bboxes.T)
    gl_c = gt_labels.reshape(_G, 1)
    oh = (gl_c == jnp.arange(_NC, dtype=jnp.int32)[None, :]).astype(jnp.float32)

    out_shape = [
        jax.ShapeDtypeStruct((_C, 1, _R), jnp.int32),
        jax.ShapeDtypeStruct((_C, 1, _R), jnp.float32),
        jax.ShapeDtypeStruct((_C, 1, _R), jnp.int32),
    ]
    scratch = [
        pltpu.VMEM((_C, _G, _R), jnp.int32),
        pltpu.VMEM((_C, _G, _R), jnp.int32),
        pltpu.VMEM((_C, 1, _R), jnp.int32),
        pltpu.VMEM((_C, 1, _R), jnp.float32),
        pltpu.VMEM((_C, 1, _R), jnp.int32),
        pltpu.VMEM((_C, 1, _R), jnp.float32),
    ]
    a, o, l = pl.pallas_call(
        _body,
        out_shape=out_shape,
        scratch_shapes=scratch,
    )(ps, pc, st, pb, gt_bboxes, gl_c, oh)
    return (a.reshape(_NP)[:_N], o.reshape(_NP)[:_N], l.reshape(_NP)[:_N])


# speedup vs baseline: 2.4996x; 1.0260x over previous
import jax
import jax.numpy as jnp
from jax import lax
from jax.experimental import pallas as pl
from jax.experimental.pallas import tpu as pltpu

_EPS = 1e-07
_INF = 100000000.0
_RADIUS = 3.0
_IOU_W = 3.0
_K = 13
_N = 20000
_G = 128
_NC = 80
_R = 1024
_NP = 20480
_C = _NP // _R
_LN10 = 2.302585092994046
_IMAX = 2147483647
_MSK = -2147483648


def _body(ps_ref, pc_ref, st_ref, pb_ref, gb_ref, gl_ref, oh_ref,
          o_gt_ref, o_ov_ref, o_lb_ref,
          key_s, iou_s, cnt_s, mpi_s, ra_s, ri_s):
    gt_i = lax.broadcasted_iota(jnp.int32, (_G, 1), 0)
    lane = lax.broadcasted_iota(jnp.int32, (1, _R), 1)

    gx1 = gb_ref[:, 0:1]
    gy1 = gb_ref[:, 1:2]
    gx2 = gb_ref[:, 2:3]
    gy2 = gb_ref[:, 3:4]
    pad = (gx1 + gy1 + gx2 + gy2) > 0.0
    gcx = (gx1 + gx2) * 0.5
    gcy = (gy1 + gy2) * 0.5
    garea = (gx2 - gx1) * (gy2 - gy1)
    gl_c = gl_ref[:, 0:1]
    oh = oh_ref[...]

    def p1(j, _):
        cx = pc_ref[j][0:1, :]
        cy = pc_ref[j][1:2, :]
        stride = st_ref[j][0:1, :]
        pb = pb_ref[j]
        px1 = pb[0:1, :]
        py1 = pb[1:2, :]
        px2 = pb[2:3, :]
        py2 = pb[3:4, :]

        mind = jnp.minimum(jnp.minimum(cx - gx1, cy - gy1),
                           jnp.minimum(gx2 - cx, gy2 - cy))
        inm = jnp.logical_and(mind > 0.0, pad)
        valid = jnp.sum(inm.astype(jnp.float32), axis=0, keepdims=True) > 0.0

        dx = cx - gcx
        dy = cy - gcy
        dist = jnp.sqrt(dx * dx + dy * dy) / stride
        dist = dist * valid.astype(jnp.float32)
        scp = jnp.exp(_LN10 * (dist - _RADIUS))

        parea = (px2 - px1) * (py2 - py1)
        iw = jnp.maximum(jnp.minimum(px2, gx2) - jnp.maximum(px1, gx1), 0.0)
        ih = jnp.maximum(jnp.minimum(py2, gy2) - jnp.maximum(py1, gy1), 0.0)
        ov = iw * ih
        iou = ov / jnp.maximum(parea + garea - ov, _EPS)

        pps = lax.dot_general(oh, ps_ref[j], (((1,), (0,)), ((), ())),
                              precision=lax.Precision.HIGHEST,
                              preferred_element_type=jnp.float32)
        sig = 1.0 / (1.0 + jnp.exp(-pps))
        sf = iou - sig
        bce = (jnp.maximum(pps, 0.0) - pps * iou
               + jnp.log1p(jnp.exp(-jnp.abs(pps))))
        cost = bce * (sf * sf) - jnp.log(iou + _EPS) * _IOU_W + scp
        cost = jnp.where(valid, cost, _INF)

        key = lax.bitcast_convert_type(cost, jnp.int32)
        key = jnp.where(lane + j * _R < _N, key, _IMAX)
        key_s[j] = key
        iou_s[j] = lax.bitcast_convert_type(iou, jnp.int32)

        rm = jnp.min(cost, axis=0, keepdims=True)
        ra = jnp.min(jnp.where(cost == rm, gt_i, _G), axis=0, keepdims=True)
        ra_s[j] = ra
        ri_s[j] = jnp.sum(jnp.where(gt_i == ra, iou, 0.0),
                          axis=0, keepdims=True)
        cnt_s[j] = jnp.zeros((1, _R), jnp.int32)
        mpi_s[j] = jnp.zeros((1, _R), jnp.float32)
        return 0

    lax.fori_loop(0, _C, p1, 0)

    def cmax0(j, m):
        return jnp.maximum(m, jnp.max(iou_s[j], axis=1, keepdims=True))
    colmax0 = lax.fori_loop(0, _C, cmax0, jnp.full((_G, 1), _MSK, jnp.int32))

    def iou_t(t, carry):
        acc, taken, colmax = carry

        def msk(j, cm):
            c, m = cm
            x = iou_s[j]
            eqm = x == colmax
            x2 = jnp.where(eqm, x | _MSK, x)
            iou_s[j] = x2
            c = c + jnp.sum(eqm.astype(jnp.int32), axis=1, keepdims=True)
            m = jnp.maximum(m, jnp.max(x2, axis=1, keepdims=True))
            return (c, m)
        count, nxt = lax.fori_loop(0, _C, msk,
                                   (jnp.zeros((_G, 1), jnp.int32),
                                    jnp.full((_G, 1), _MSK, jnp.int32)))

        m = jnp.minimum(count, jnp.maximum(_K - taken, 0))
        v = jnp.maximum(lax.bitcast_convert_type(colmax, jnp.float32), 0.0)
        return (acc + v * m.astype(jnp.float32), taken + m, nxt)

    acc, _, _ = lax.fori_loop(0, _K, iou_t,
                              (jnp.zeros((_G, 1), jnp.float32),
                               jnp.zeros((_G, 1), jnp.int32), colmax0))
    dk = jnp.maximum(acc.astype(jnp.int32), 1)

    def unmask(j, _):
        iou_s[j] = iou_s[j] & _IMAX
        return 0
    lax.fori_loop(0, _C, unmask, 0)

    def _minarg(j, mr, k):
        m, r = mr
        mj = jnp.min(k, axis=1, keepdims=True)
        cand = jnp.min(jnp.where(k == mj, lane + j * _R, _NP),
                       axis=1, keepdims=True)
        r = jnp.where(mj < m, cand,
                      jnp.where(mj == m, jnp.minimum(r, cand), r))
        return (jnp.minimum(m, mj), r)

    _mr0 = (jnp.full((_G, 1), _IMAX, jnp.int32),
            jnp.full((_G, 1), _NP, jnp.int32))

    def ma0(j, mr):
        return _minarg(j, mr, key_s[j])
    _, ridx0 = lax.fori_loop(0, _C, ma0, _mr0)

    def cost_t(t, ridx):
        sel = t < dk
        gpack = jnp.where(sel, 2048 + gt_i, 0)

        def upd(j, mr):
            ind = (lane + j * _R) == ridx
            k2 = jnp.where(ind, _IMAX, key_s[j])
            key_s[j] = k2
            cnt_s[j] = cnt_s[j] + jnp.sum(jnp.where(ind, gpack, 0),
                                          axis=0, keepdims=True)
            iouf = lax.bitcast_convert_type(iou_s[j], jnp.float32)
            msel = jnp.logical_and(ind, sel)
            mpi_s[j] = mpi_s[j] + jnp.sum(jnp.where(msel, iouf, 0.0),
                                          axis=0, keepdims=True)
            return _minarg(j, mr, k2)
        _, ridx_next = lax.fori_loop(0, _C, upd, _mr0)
        return ridx_next

    lax.fori_loop(0, jnp.max(dk), cost_t, ridx0)

    def fin(j, _):
        a = cnt_s[j]
        c = a >> 11
        fg = c > 0
        multi = c > 1
        mg = jnp.where(multi, ra_s[j], a & 2047)
        ovl = jnp.where(multi, ri_s[j], mpi_s[j])
        lab = jnp.sum(jnp.where(gt_i == mg, gl_c, 0), axis=0, keepdims=True)
        o_gt_ref[j] = jnp.where(fg, mg + 1, 0)
        o_ov_ref[j] = jnp.where(fg, ovl, -_INF)
        o_lb_ref[j] = jnp.where(fg, lab, -1)
        return 0

    lax.fori_loop(0, _C, fin, 0)


def kernel(pred_scores, priors, pred_bboxes, gt_bboxes, gt_labels):
    padn = _NP - _N

    def chunked(xt):
        xt = jnp.pad(xt, ((0, 0), (0, padn)))
        k = xt.shape[0]
        return xt.reshape(k, _C, _R).transpose(1, 0, 2)

    ps = chunked(pred_scores.T)
    pc = chunked(priors[:, :2].T)
    st = chunked(priors[:, 2:3].T)
    pb = chunked(pred_bboxes.T)
    gl_c = gt_labels.reshape(_G, 1)
    oh = (gl_c == jnp.arange(_NC, dtype=jnp.int32)[None, :]).astype(jnp.float32)

    out_shape = [
        jax.ShapeDtypeStruct((_C, 1, _R), jnp.int32),
        jax.ShapeDtypeStruct((_C, 1, _R), jnp.float32),
        jax.ShapeDtypeStruct((_C, 1, _R), jnp.int32),
    ]
    scratch = [
        pltpu.VMEM((_C, _G, _R), jnp.int32),
        pltpu.VMEM((_C, _G, _R), jnp.int32),
        pltpu.VMEM((_C, 1, _R), jnp.int32),
        pltpu.VMEM((_C, 1, _R), jnp.float32),
        pltpu.VMEM((_C, 1, _R), jnp.int32),
        pltpu.VMEM((_C, 1, _R), jnp.float32),
    ]
    a, o, l = pl.pallas_call(
        _body,
        out_shape=out_shape,
        scratch_shapes=scratch,
    )(ps, pc, st, pb, gt_bboxes, gl_c, oh)
    return (a.reshape(_NP)[:_N], o.reshape(_NP)[:_N], l.reshape(_NP)[:_N])


# speedup vs baseline: 2.9648x; 1.1861x over previous
import jax
import jax.numpy as jnp
from jax import lax
from jax.experimental import pallas as pl
from jax.experimental.pallas import tpu as pltpu

_EPS = 1e-07
_INF = 100000000.0
_RADIUS = 3.0
_IOU_W = 3.0
_K = 13
_N = 20000
_G = 128
_NC = 80
_R = 2048
_NP = 20480
_C = _NP // _R
_LN10 = 2.302585092994046
_IMAX = 2147483647
_MSK = -2147483648


def _body(ps_ref, pc_ref, st_ref, pb_ref, gb_ref, gl_ref, oh_ref,
          o_gt_ref, o_ov_ref, o_lb_ref,
          key_s, iou_s, cnt_s, mpi_s, ra_s, ri_s):
    gt_i = lax.broadcasted_iota(jnp.int32, (_G, 1), 0)
    lane = lax.broadcasted_iota(jnp.int32, (1, _R), 1)

    gx1 = gb_ref[:, 0:1]
    gy1 = gb_ref[:, 1:2]
    gx2 = gb_ref[:, 2:3]
    gy2 = gb_ref[:, 3:4]
    pad = (gx1 + gy1 + gx2 + gy2) > 0.0
    gcx = (gx1 + gx2) * 0.5
    gcy = (gy1 + gy2) * 0.5
    garea = (gx2 - gx1) * (gy2 - gy1)
    gl_c = gl_ref[:, 0:1]
    oh = oh_ref[...]

    def p1(j, _):
        cx = pc_ref[j][0:1, :]
        cy = pc_ref[j][1:2, :]
        stride = st_ref[j][0:1, :]
        pb = pb_ref[j]
        px1 = pb[0:1, :]
        py1 = pb[1:2, :]
        px2 = pb[2:3, :]
        py2 = pb[3:4, :]

        mind = jnp.minimum(jnp.minimum(cx - gx1, cy - gy1),
                           jnp.minimum(gx2 - cx, gy2 - cy))
        inm = jnp.logical_and(mind > 0.0, pad)
        valid = jnp.sum(inm.astype(jnp.float32), axis=0, keepdims=True) > 0.0

        dx = cx - gcx
        dy = cy - gcy
        dist = jnp.sqrt(dx * dx + dy * dy) / stride
        dist = dist * valid.astype(jnp.float32)
        scp = jnp.exp(_LN10 * (dist - _RADIUS))

        parea = (px2 - px1) * (py2 - py1)
        iw = jnp.maximum(jnp.minimum(px2, gx2) - jnp.maximum(px1, gx1), 0.0)
        ih = jnp.maximum(jnp.minimum(py2, gy2) - jnp.maximum(py1, gy1), 0.0)
        ov = iw * ih
        iou = ov / jnp.maximum(parea + garea - ov, _EPS)

        pps = lax.dot_general(oh, ps_ref[j], (((1,), (0,)), ((), ())),
                              precision=lax.Precision.HIGHEST,
                              preferred_element_type=jnp.float32)
        sig = 1.0 / (1.0 + jnp.exp(-pps))
        sf = iou - sig
        bce = (jnp.maximum(pps, 0.0) - pps * iou
               + jnp.log1p(jnp.exp(-jnp.abs(pps))))
        cost = bce * (sf * sf) - jnp.log(iou + _EPS) * _IOU_W + scp
        cost = jnp.where(valid, cost, _INF)

        key = lax.bitcast_convert_type(cost, jnp.int32)
        key = jnp.where(lane + j * _R < _N, key, _IMAX)
        key_s[j] = key
        iou_s[j] = lax.bitcast_convert_type(iou, jnp.int32)

        rm = jnp.min(cost, axis=0, keepdims=True)
        ra = jnp.min(jnp.where(cost == rm, gt_i, _G), axis=0, keepdims=True)
        ra_s[j] = ra
        ri_s[j] = jnp.sum(jnp.where(gt_i == ra, iou, 0.0),
                          axis=0, keepdims=True)
        cnt_s[j] = jnp.zeros((1, _R), jnp.int32)
        mpi_s[j] = jnp.zeros((1, _R), jnp.float32)
        return 0

    lax.fori_loop(0, _C, p1, 0)

    def cmax0(j, m):
        return jnp.maximum(m, jnp.max(iou_s[j], axis=1, keepdims=True))
    colmax0 = lax.fori_loop(0, _C, cmax0, jnp.full((_G, 1), _MSK, jnp.int32))

    def iou_t(t, carry):
        acc, taken, colmax = carry

        def msk(j, cm):
            c, m = cm
            x = iou_s[j]
            eqm = x == colmax
            x2 = jnp.where(eqm, x | _MSK, x)
            iou_s[j] = x2
            c = c + jnp.sum(eqm.astype(jnp.int32), axis=1, keepdims=True)
            m = jnp.maximum(m, jnp.max(x2, axis=1, keepdims=True))
            return (c, m)
        count, nxt = lax.fori_loop(0, _C, msk,
                                   (jnp.zeros((_G, 1), jnp.int32),
                                    jnp.full((_G, 1), _MSK, jnp.int32)))

        m = jnp.minimum(count, jnp.maximum(_K - taken, 0))
        v = jnp.maximum(lax.bitcast_convert_type(colmax, jnp.float32), 0.0)
        return (acc + v * m.astype(jnp.float32), taken + m, nxt)

    acc, _, _ = lax.fori_loop(0, _K, iou_t,
                              (jnp.zeros((_G, 1), jnp.float32),
                               jnp.zeros((_G, 1), jnp.int32), colmax0))
    dk = jnp.maximum(acc.astype(jnp.int32), 1)

    def unmask(j, _):
        iou_s[j] = iou_s[j] & _IMAX
        return 0
    lax.fori_loop(0, _C, unmask, 0)

    def _minarg(j, mr, k):
        m, r = mr
        mj = jnp.min(k, axis=1, keepdims=True)
        cand = jnp.min(jnp.where(k == mj, lane + j * _R, _NP),
                       axis=1, keepdims=True)
        r = jnp.where(mj < m, cand,
                      jnp.where(mj == m, jnp.minimum(r, cand), r))
        return (jnp.minimum(m, mj), r)

    _mr0 = (jnp.full((_G, 1), _IMAX, jnp.int32),
            jnp.full((_G, 1), _NP, jnp.int32))

    def ma0(j, mr):
        return _minarg(j, mr, key_s[j])
    _, ridx0 = lax.fori_loop(0, _C, ma0, _mr0)

    def cost_t(t, ridx):
        sel = t < dk
        gpack = jnp.where(sel, 2048 + gt_i, 0)

        def upd(j, mr):
            ind = (lane + j * _R) == ridx
            k2 = jnp.where(ind, _IMAX, key_s[j])
            key_s[j] = k2
            cnt_s[j] = cnt_s[j] + jnp.sum(jnp.where(ind, gpack, 0),
                                          axis=0, keepdims=True)
            iouf = lax.bitcast_convert_type(iou_s[j], jnp.float32)
            msel = jnp.logical_and(ind, sel)
            mpi_s[j] = mpi_s[j] + jnp.sum(jnp.where(msel, iouf, 0.0),
                                          axis=0, keepdims=True)
            return _minarg(j, mr, k2)
        _, ridx_next = lax.fori_loop(0, _C, upd, _mr0)
        return ridx_next

    lax.fori_loop(0, jnp.max(dk), cost_t, ridx0)

    def fin(j, _):
        a = cnt_s[j]
        c = a >> 11
        fg = c > 0
        multi = c > 1
        mg = jnp.where(multi, ra_s[j], a & 2047)
        ovl = jnp.where(multi, ri_s[j], mpi_s[j])
        lab = jnp.sum(jnp.where(gt_i == mg, gl_c, 0), axis=0, keepdims=True)
        o_gt_ref[j] = jnp.where(fg, mg + 1, 0)
        o_ov_ref[j] = jnp.where(fg, ovl, -_INF)
        o_lb_ref[j] = jnp.where(fg, lab, -1)
        return 0

    lax.fori_loop(0, _C, fin, 0)


def kernel(pred_scores, priors, pred_bboxes, gt_bboxes, gt_labels):
    padn = _NP - _N

    def chunked(xt):
        xt = jnp.pad(xt, ((0, 0), (0, padn)))
        k = xt.shape[0]
        return xt.reshape(k, _C, _R).transpose(1, 0, 2)

    ps = chunked(pred_scores.T)
    pc = chunked(priors[:, :2].T)
    st = chunked(priors[:, 2:3].T)
    pb = chunked(pred_bboxes.T)
    gl_c = gt_labels.reshape(_G, 1)
    oh = (gl_c == jnp.arange(_NC, dtype=jnp.int32)[None, :]).astype(jnp.float32)

    out_shape = [
        jax.ShapeDtypeStruct((_C, 1, _R), jnp.int32),
        jax.ShapeDtypeStruct((_C, 1, _R), jnp.float32),
        jax.ShapeDtypeStruct((_C, 1, _R), jnp.int32),
    ]
    scratch = [
        pltpu.VMEM((_C, _G, _R), jnp.int32),
        pltpu.VMEM((_C, _G, _R), jnp.int32),
        pltpu.VMEM((_C, 1, _R), jnp.int32),
        pltpu.VMEM((_C, 1, _R), jnp.float32),
        pltpu.VMEM((_C, 1, _R), jnp.int32),
        pltpu.VMEM((_C, 1, _R), jnp.float32),
    ]
    a, o, l = pl.pallas_call(
        _body,
        out_shape=out_shape,
        scratch_shapes=scratch,
    )(ps, pc, st, pb, gt_bboxes, gl_c, oh)
    return (a.reshape(_NP)[:_N], o.reshape(_NP)[:_N], l.reshape(_NP)[:_N])


# speedup vs baseline: 3.0765x; 1.0377x over previous
import jax
import jax.numpy as jnp
from jax import lax
from jax.experimental import pallas as pl
from jax.experimental.pallas import tpu as pltpu

_EPS = 1e-07
_INF = 100000000.0
_RADIUS = 3.0
_IOU_W = 3.0
_K = 13
_N = 20000
_G = 128
_NC = 80
_R = 4096
_NP = 20480
_C = _NP // _R
_LN10 = 2.302585092994046
_IMAX = 2147483647
_MSK = -2147483648


def _body(ps_ref, pc_ref, st_ref, pb_ref, gb_ref, gl_ref, oh_ref,
          o_gt_ref, o_ov_ref, o_lb_ref,
          key_s, iou_s, cnt_s, mpi_s, ra_s, ri_s):
    gt_i = lax.broadcasted_iota(jnp.int32, (_G, 1), 0)
    lane = lax.broadcasted_iota(jnp.int32, (1, _R), 1)

    gx1 = gb_ref[:, 0:1]
    gy1 = gb_ref[:, 1:2]
    gx2 = gb_ref[:, 2:3]
    gy2 = gb_ref[:, 3:4]
    pad = (gx1 + gy1 + gx2 + gy2) > 0.0
    gcx = (gx1 + gx2) * 0.5
    gcy = (gy1 + gy2) * 0.5
    garea = (gx2 - gx1) * (gy2 - gy1)
    gl_c = gl_ref[:, 0:1]
    oh = oh_ref[...]

    def p1(j, _):
        cx = pc_ref[j][0:1, :]
        cy = pc_ref[j][1:2, :]
        stride = st_ref[j][0:1, :]
        pb = pb_ref[j]
        px1 = pb[0:1, :]
        py1 = pb[1:2, :]
        px2 = pb[2:3, :]
        py2 = pb[3:4, :]

        mind = jnp.minimum(jnp.minimum(cx - gx1, cy - gy1),
                           jnp.minimum(gx2 - cx, gy2 - cy))
        inm = jnp.logical_and(mind > 0.0, pad)
        valid = jnp.sum(inm.astype(jnp.float32), axis=0, keepdims=True) > 0.0

        dx = cx - gcx
        dy = cy - gcy
        dist = jnp.sqrt(dx * dx + dy * dy) / stride
        dist = dist * valid.astype(jnp.float32)
        scp = jnp.exp(_LN10 * (dist - _RADIUS))

        parea = (px2 - px1) * (py2 - py1)
        iw = jnp.maximum(jnp.minimum(px2, gx2) - jnp.maximum(px1, gx1), 0.0)
        ih = jnp.maximum(jnp.minimum(py2, gy2) - jnp.maximum(py1, gy1), 0.0)
        ov = iw * ih
        iou = ov / jnp.maximum(parea + garea - ov, _EPS)

        pps = lax.dot_general(oh, ps_ref[j], (((1,), (0,)), ((), ())),
                              precision=lax.Precision.HIGHEST,
                              preferred_element_type=jnp.float32)
        sig = 1.0 / (1.0 + jnp.exp(-pps))
        sf = iou - sig
        bce = (jnp.maximum(pps, 0.0) - pps * iou
               + jnp.log1p(jnp.exp(-jnp.abs(pps))))
        cost = bce * (sf * sf) - jnp.log(iou + _EPS) * _IOU_W + scp
        cost = jnp.where(valid, cost, _INF)

        key = lax.bitcast_convert_type(cost, jnp.int32)
        key = jnp.where(lane + j * _R < _N, key, _IMAX)
        key_s[j] = key
        iou_s[j] = lax.bitcast_convert_type(iou, jnp.int32)

        rm = jnp.min(cost, axis=0, keepdims=True)
        ra = jnp.min(jnp.where(cost == rm, gt_i, _G), axis=0, keepdims=True)
        ra_s[j] = ra
        ri_s[j] = jnp.sum(jnp.where(gt_i == ra, iou, 0.0),
                          axis=0, keepdims=True)
        cnt_s[j] = jnp.zeros((1, _R), jnp.int32)
        mpi_s[j] = jnp.zeros((1, _R), jnp.float32)
        return 0

    lax.fori_loop(0, _C, p1, 0)

    def cmax0(j, m):
        return jnp.maximum(m, jnp.max(iou_s[j], axis=1, keepdims=True))
    colmax0 = lax.fori_loop(0, _C, cmax0, jnp.full((_G, 1), _MSK, jnp.int32))

    def iou_t(t, carry):
        acc, taken, colmax = carry

        def msk(j, cm):
            c, m = cm
            x = iou_s[j]
            eqm = x == colmax
            x2 = jnp.where(eqm, x | _MSK, x)
            iou_s[j] = x2
            c = c + jnp.sum(eqm.astype(jnp.int32), axis=1, keepdims=True)
            m = jnp.maximum(m, jnp.max(x2, axis=1, keepdims=True))
            return (c, m)
        count, nxt = lax.fori_loop(0, _C, msk,
                                   (jnp.zeros((_G, 1), jnp.int32),
                                    jnp.full((_G, 1), _MSK, jnp.int32)))

        m = jnp.minimum(count, jnp.maximum(_K - taken, 0))
        v = jnp.maximum(lax.bitcast_convert_type(colmax, jnp.float32), 0.0)
        return (acc + v * m.astype(jnp.float32), taken + m, nxt)

    acc, _, _ = lax.fori_loop(0, _K, iou_t,
                              (jnp.zeros((_G, 1), jnp.float32),
                               jnp.zeros((_G, 1), jnp.int32), colmax0))
    dk = jnp.maximum(acc.astype(jnp.int32), 1)

    def unmask(j, _):
        iou_s[j] = iou_s[j] & _IMAX
        return 0
    lax.fori_loop(0, _C, unmask, 0)

    def _minarg(j, mr, k):
        m, r = mr
        mj = jnp.min(k, axis=1, keepdims=True)
        cand = jnp.min(jnp.where(k == mj, lane + j * _R, _NP),
                       axis=1, keepdims=True)
        r = jnp.where(mj < m, cand,
                      jnp.where(mj == m, jnp.minimum(r, cand), r))
        return (jnp.minimum(m, mj), r)

    _mr0 = (jnp.full((_G, 1), _IMAX, jnp.int32),
            jnp.full((_G, 1), _NP, jnp.int32))

    def ma0(j, mr):
        return _minarg(j, mr, key_s[j])
    _, ridx0 = lax.fori_loop(0, _C, ma0, _mr0)

    def cost_t(t, ridx):
        sel = t < dk
        gpack = jnp.where(sel, 2048 + gt_i, 0)

        def upd(j, mr):
            ind = (lane + j * _R) == ridx
            k2 = jnp.where(ind, _IMAX, key_s[j])
            key_s[j] = k2
            cnt_s[j] = cnt_s[j] + jnp.sum(jnp.where(ind, gpack, 0),
                                          axis=0, keepdims=True)
            iouf = lax.bitcast_convert_type(iou_s[j], jnp.float32)
            msel = jnp.logical_and(ind, sel)
            mpi_s[j] = mpi_s[j] + jnp.sum(jnp.where(msel, iouf, 0.0),
                                          axis=0, keepdims=True)
            return _minarg(j, mr, k2)
        _, ridx_next = lax.fori_loop(0, _C, upd, _mr0)
        return ridx_next

    lax.fori_loop(0, jnp.max(dk), cost_t, ridx0)

    def fin(j, _):
        a = cnt_s[j]
        c = a >> 11
        fg = c > 0
        multi = c > 1
        mg = jnp.where(multi, ra_s[j], a & 2047)
        ovl = jnp.where(multi, ri_s[j], mpi_s[j])
        lab = jnp.sum(jnp.where(gt_i == mg, gl_c, 0), axis=0, keepdims=True)
        o_gt_ref[j] = jnp.where(fg, mg + 1, 0)
        o_ov_ref[j] = jnp.where(fg, ovl, -_INF)
        o_lb_ref[j] = jnp.where(fg, lab, -1)
        return 0

    lax.fori_loop(0, _C, fin, 0)


def kernel(pred_scores, priors, pred_bboxes, gt_bboxes, gt_labels):
    padn = _NP - _N

    def chunked(xt):
        xt = jnp.pad(xt, ((0, 0), (0, padn)))
        k = xt.shape[0]
        return xt.reshape(k, _C, _R).transpose(1, 0, 2)

    ps = chunked(pred_scores.T)
    pc = chunked(priors[:, :2].T)
    st = chunked(priors[:, 2:3].T)
    pb = chunked(pred_bboxes.T)
    gl_c = gt_labels.reshape(_G, 1)
    oh = (gl_c == jnp.arange(_NC, dtype=jnp.int32)[None, :]).astype(jnp.float32)

    out_shape = [
        jax.ShapeDtypeStruct((_C, 1, _R), jnp.int32),
        jax.ShapeDtypeStruct((_C, 1, _R), jnp.float32),
        jax.ShapeDtypeStruct((_C, 1, _R), jnp.int32),
    ]
    scratch = [
        pltpu.VMEM((_C, _G, _R), jnp.int32),
        pltpu.VMEM((_C, _G, _R), jnp.int32),
        pltpu.VMEM((_C, 1, _R), jnp.int32),
        pltpu.VMEM((_C, 1, _R), jnp.float32),
        pltpu.VMEM((_C, 1, _R), jnp.int32),
        pltpu.VMEM((_C, 1, _R), jnp.float32),
    ]
    a, o, l = pl.pallas_call(
        _body,
        out_shape=out_shape,
        scratch_shapes=scratch,
    )(ps, pc, st, pb, gt_bboxes, gl_c, oh)
    return (a.reshape(_NP)[:_N], o.reshape(_NP)[:_N], l.reshape(_NP)[:_N])
